# baseline (speedup 1.0000x reference)
import functools

import jax
import jax.numpy as jnp
from jax import lax
from jax.experimental import pallas as pl
from jax.experimental.pallas import tpu as pltpu
from jax.experimental.pallas import tpu_sc as plsc

NC = 2
NS = 16
NW = NC * NS
L = 16

N = 10000
D = 128
B = 4096
MAXNB = 32
K0 = 25
K1 = 10
H = 256

_MESH = plsc.VectorSubcoreMesh(
    core_axis_name="c", subcore_axis_name="s", num_cores=NC, num_subcores=NS
)
_PARAMS = pltpu.CompilerParams(
    needs_layout_passes=False, use_tc_tiling_on_sc=False
)


def _wid():
  return lax.axis_index("s") * NC + lax.axis_index("c")


NPW = 320
SPACING = 320
CH = 4
STRIDE_A = K0 + 1
ROWS_A = CH * STRIDE_A
NCH_A = NPW // CH
NBLK = D // 32


def _agg_all_body(featsb_hbm, adj_hbm, out_hbm, adj_v, idx_c, buf0, buf1,
                  stage, sem0, sem1):
  wid = _wid()
  base = pl.multiple_of(jnp.minimum(wid * SPACING, N - NPW), 8)
  pltpu.sync_copy(adj_hbm.at[pl.ds(base, NPW)], adj_v)

  iota = lax.iota(jnp.int32, L)
  msk9 = iota < (K0 - L)

  @pl.loop(0, NPW)
  def _(i):
    v0 = adj_v[i, pl.ds(0, L)]
    v1 = adj_v[i, pl.ds(L, L)]
    b26 = i * STRIDE_A
    plsc.store_scatter(idx_c, [b26 + iota], v0)
    plsc.store_scatter(idx_c, [b26 + L + iota], v1, mask=msk9)
    plsc.store_scatter(idx_c, [jnp.zeros((L,), jnp.int32) + (b26 + K0)], v0)

  def start(g, buf, sem):
    pltpu.async_copy(featsb_hbm.at[idx_c.at[pl.ds(g * ROWS_A, ROWS_A)]],
                     buf, sem)

  def wait(buf, sem):
    pltpu.make_async_copy(featsb_hbm.at[pl.ds(0, ROWS_A)], buf, sem).wait()

  inv = jnp.float32(1.0 / K0)

  def compute(g, buf):
    for t in range(CH):
      node = g * CH + t
      row0 = STRIDE_A * t

      def load_row(row):
        out = []
        for kb in range(NBLK):
          a, b = plsc.unpack(buf[row, pl.ds(32 * kb, 32)],
                             format=plsc.PackFormat.INTERLEAVED)
          out.extend((a, b))
        return tuple(out)

      accs = load_row(row0)

      def body(r, accs, _row0=row0):
        vs = load_row(_row0 + r)
        return tuple(a + v for a, v in zip(accs, vs))

      accs = pl.loop(1, K0, init_carry=accs, unroll=4)(body)
      for kb in range(NBLK):
        stage[node, pl.ds(32 * kb, 32)] = plsc.pack(
            accs[2 * kb] * inv, accs[2 * kb + 1] * inv,
            format=plsc.PackFormat.INTERLEAVED)

  start(0, buf0, sem0)

  @pl.loop(0, NCH_A // 2)
  def _(gp):
    g0 = gp * 2
    start(g0 + 1, buf1, sem1)
    wait(buf0, sem0)
    compute(g0, buf0)
    start((g0 + 2) % NCH_A, buf0, sem0)
    wait(buf1, sem1)
    compute(g0 + 1, buf1)

  wait(buf0, sem0)
  pltpu.sync_copy(stage, out_hbm.at[pl.ds(base, NPW)])


_agg_all = functools.partial(
    pl.kernel,
    out_type=jax.ShapeDtypeStruct((N, D), jnp.bfloat16),
    mesh=_MESH,
    compiler_params=_PARAMS,
    scratch_types=[
        pltpu.VMEM((NPW, MAXNB), jnp.int32),
        pltpu.VMEM((NPW * STRIDE_A,), jnp.int32),
        pltpu.VMEM((ROWS_A, D), jnp.bfloat16),
        pltpu.VMEM((ROWS_A, D), jnp.bfloat16),
        pltpu.VMEM((NPW, D), jnp.bfloat16),
        pltpu.SemaphoreType.DMA,
        pltpu.SemaphoreType.DMA,
    ],
)(_agg_all_body)


JB = B // NW
NCH_B = B // L
G_C = 8
KIDX = K1 * G_C
NG_C = JB // G_C
HBLK = H // 32


def _batch_body(x_hbm, nodes_hbm, adj_hbm, aggall_hbm, hnode_hbm,
                xdedup_hbm, agg0_hbm, agg1_hbm,
                nodes_v, dbuf, prevscr, first_v, xrows, adj_rows, idx_c,
                rows0, rows1, agg0buf, agg1st,
                semx, sema, semb, sem0, sem1):
  wid = _wid()
  jbase = pl.multiple_of(wid * JB, 8)
  pltpu.sync_copy(nodes_hbm, nodes_v)
  ndslice = nodes_v.at[pl.ds(jbase, JB)]
  pltpu.async_copy(adj_hbm.at[ndslice], adj_rows, sema)
  pltpu.async_copy(aggall_hbm.at[ndslice], agg0buf, semb)

  iota = lax.iota(jnp.int32, L)
  prevscr[pl.ds(0, L)] = jnp.full((L,), -1, jnp.int32)

  @pl.loop(0, NCH_B)
  def _(i):
    c = (NCH_B - 1) - i
    j = c * L + iota
    nd = nodes_v[pl.ds(c * L, L)]
    key = nd * B + j
    sk, sv = plsc.sort_key_val(key, j)
    snode = lax.shift_right_logical(sk, 12)
    plsc.store_scatter(prevscr, [iota + 1], snode)
    prev = prevscr[pl.ds(0, L)]
    head = (iota == 0) | (snode != prev)
    plsc.store_scatter(dbuf, [snode], sv, mask=head)

  for q in range(JB // L):
    nd16 = nodes_v[pl.ds(jbase + q * L, L)]
    first_v[pl.ds(q * L, L)] = plsc.load_gather(dbuf, [nd16])
  pltpu.async_copy(x_hbm.at[first_v], xrows, semx)

  msk10 = iota < K1
  pltpu.make_async_copy(adj_hbm.at[pl.ds(0, JB)], adj_rows, sema).wait()

  @pl.loop(0, JB)
  def _(j):
    v = adj_rows[j, pl.ds(0, L)]
    plsc.store_scatter(idx_c, [j * K1 + iota], v, mask=msk10)

  pltpu.make_async_copy(aggall_hbm.at[pl.ds(0, JB)], agg0buf, semb).wait()
  pltpu.sync_copy(agg0buf, agg0_hbm.at[pl.ds(jbase, JB)])
  pltpu.make_async_copy(x_hbm.at[pl.ds(0, JB)], xrows, semx).wait()
  pltpu.sync_copy(xrows, xdedup_hbm.at[pl.ds(jbase, JB)])

  inv = jnp.float32(1.0 / K1)

  def start(g, buf, sem_):
    pltpu.async_copy(hnode_hbm.at[idx_c.at[pl.ds(g * KIDX, KIDX)]],
                     buf, sem_)

  def wait(buf, sem_):
    pltpu.make_async_copy(hnode_hbm.at[pl.ds(0, KIDX)], buf, sem_).wait()

  def compute(g, buf):
    for t in range(G_C):
      j = g * G_C + t
      row0 = t * K1

      def load_row(row):
        out = []
        for kb in range(HBLK):
          a, b = plsc.unpack(buf[row, pl.ds(32 * kb, 32)],
                             format=plsc.PackFormat.INTERLEAVED)
          out.extend((a, b))
        return tuple(out)

      accs = load_row(row0)

      def body(r, accs, _row0=row0):
        vs = load_row(_row0 + r)
        return tuple(a + v for a, v in zip(accs, vs))

      accs = pl.loop(1, K1, init_carry=accs, unroll=3)(body)
      for kb in range(HBLK):
        agg1st[j, pl.ds(32 * kb, 32)] = plsc.pack(
            accs[2 * kb] * inv, accs[2 * kb + 1] * inv,
            format=plsc.PackFormat.INTERLEAVED)

  start(0, rows0, sem0)

  @pl.loop(0, NG_C // 2)
  def _(gp):
    g0 = gp * 2
    start(g0 + 1, rows1, sem1)
    wait(rows0, sem0)
    compute(g0, rows0)
    start((g0 + 2) % NG_C, rows0, sem0)
    wait(rows1, sem1)
    compute(g0 + 1, rows1)

  wait(rows0, sem0)
  pltpu.sync_copy(agg1st, agg1_hbm.at[pl.ds(jbase, JB)])


_batch = functools.partial(
    pl.kernel,
    out_type=(jax.ShapeDtypeStruct((B, D), jnp.float32),
              jax.ShapeDtypeStruct((B, D), jnp.bfloat16),
              jax.ShapeDtypeStruct((B, H), jnp.bfloat16)),
    mesh=_MESH,
    compiler_params=_PARAMS,
    scratch_types=[
        pltpu.VMEM((B,), jnp.int32),
        pltpu.VMEM((N,), jnp.int32),
        pltpu.VMEM((2 * L,), jnp.int32),
        pltpu.VMEM((JB,), jnp.int32),
        pltpu.VMEM((JB, D), jnp.float32),
        pltpu.VMEM((JB, MAXNB), jnp.int32),
        pltpu.VMEM((JB * K1,), jnp.int32),
        pltpu.VMEM((KIDX, H), jnp.bfloat16),
        pltpu.VMEM((KIDX, H), jnp.bfloat16),
        pltpu.VMEM((JB, D), jnp.bfloat16),
        pltpu.VMEM((JB, H), jnp.bfloat16),
        pltpu.SemaphoreType.DMA,
        pltpu.SemaphoreType.DMA,
        pltpu.SemaphoreType.DMA,
        pltpu.SemaphoreType.DMA,
        pltpu.SemaphoreType.DMA,
    ],
)(_batch_body)


def _tc1_body(xa_ref, xb_ref, wa_ref, wb_ref, b_ref, o_ref):
  o_ref[...] = jax.nn.relu(
      jnp.dot(xa_ref[...], wa_ref[...], preferred_element_type=jnp.float32)
      + jnp.dot(xb_ref[...], wb_ref[...], preferred_element_type=jnp.float32)
      + b_ref[...]).astype(o_ref.dtype)


def _tc_linear(xa, xb, waT, wbT, bias, block_rows, out_dtype):
  rows = xa.shape[0]
  ka = xa.shape[1]
  kb = xb.shape[1]
  h = waT.shape[1]
  grid = rows // block_rows
  return pl.pallas_call(
      _tc1_body,
      grid=(grid,),
      in_specs=[
          pl.BlockSpec((block_rows, ka), lambda i: (i, 0)),
          pl.BlockSpec((block_rows, kb), lambda i: (i, 0)),
          pl.BlockSpec((ka, h), lambda i: (0, 0)),
          pl.BlockSpec((kb, h), lambda i: (0, 0)),
          pl.BlockSpec((1, h), lambda i: (0, 0)),
      ],
      out_specs=pl.BlockSpec((block_rows, h), lambda i: (i, 0)),
      out_shape=jax.ShapeDtypeStruct((rows, h), out_dtype),
  )(xa, xb, waT, wbT, bias)


def _tc2_body(xd_ref, a0_ref, a1_ref, w0a_ref, w0b_ref, b0_ref,
              w1a_ref, w1b_ref, b1_ref, o_ref):
  xd = xd_ref[...].astype(jnp.bfloat16)
  h0 = jax.nn.relu(
      jnp.dot(xd, w0a_ref[...], preferred_element_type=jnp.float32)
      + jnp.dot(a0_ref[...], w0b_ref[...], preferred_element_type=jnp.float32)
      + b0_ref[...]).astype(jnp.bfloat16)
  o_ref[...] = jax.nn.relu(
      jnp.dot(h0, w1a_ref[...], preferred_element_type=jnp.float32)
      + jnp.dot(a1_ref[...], w1b_ref[...], preferred_element_type=jnp.float32)
      + b1_ref[...])


def _tc_layer2(xd, a0, a1, w0aT, w0bT, b0r, w1aT, w1bT, b1r, block_rows):
  rows = xd.shape[0]
  grid = rows // block_rows
  return pl.pallas_call(
      _tc2_body,
      grid=(grid,),
      in_specs=[
          pl.BlockSpec((block_rows, D), lambda i: (i, 0)),
          pl.BlockSpec((block_rows, D), lambda i: (i, 0)),
          pl.BlockSpec((block_rows, H), lambda i: (i, 0)),
          pl.BlockSpec((D, H), lambda i: (0, 0)),
          pl.BlockSpec((D, H), lambda i: (0, 0)),
          pl.BlockSpec((1, H), lambda i: (0, 0)),
          pl.BlockSpec((H, H), lambda i: (0, 0)),
          pl.BlockSpec((H, H), lambda i: (0, 0)),
          pl.BlockSpec((1, H), lambda i: (0, 0)),
      ],
      out_specs=pl.BlockSpec((block_rows, H), lambda i: (i, 0)),
      out_shape=jax.ShapeDtypeStruct((rows, H), jnp.float32),
  )(xd, a0, a1, w0aT, w0bT, b0r, w1aT, w1bT, b1r)


def kernel(x, nodes, feats, adj, W0, b0, W1, b1):
  assert feats.shape == (N, D) and nodes.shape == (B,)
  assert adj.shape == (N, MAXNB)
  nodes = nodes.astype(jnp.int32)
  adj32 = adj.astype(jnp.int32)
  featsb = feats.astype(jnp.bfloat16)
  w0aT = W0[:, :D].T.astype(jnp.bfloat16)
  w0bT = W0[:, D:].T.astype(jnp.bfloat16)
  w1aT = W1[:, :H].T.astype(jnp.bfloat16)
  w1bT = W1[:, H:].T.astype(jnp.bfloat16)
  b0r = b0.reshape(1, H)
  b1r = b1.reshape(1, H)

  agg_all = _agg_all(featsb, adj32)
  h_node = _tc_linear(featsb, agg_all, w0aT, w0bT, b0r, block_rows=2000,
                      out_dtype=jnp.bfloat16)
  x_dedup, agg0, agg1 = _batch(x, nodes, adj32, agg_all, h_node)
  out = _tc_layer2(x_dedup, agg0, agg1, w0aT, w0bT, b0r,
                   w1aT, w1bT, b1r, block_rows=1024)
  return out


# speedup vs baseline: 1.0702x; 1.0702x over previous
import functools

import jax
import jax.numpy as jnp
import numpy as np
from jax import lax
from jax.experimental import pallas as pl
from jax.experimental.pallas import tpu as pltpu
from jax.experimental.pallas import tpu_sc as plsc

NC = 2
NS = 16
NW = NC * NS
L = 16

N = 10000
D = 128
B = 4096
MAXNB = 32
K0 = 25
K1 = 10
H = 256

_MESH = plsc.VectorSubcoreMesh(
    core_axis_name="c", subcore_axis_name="s", num_cores=NC, num_subcores=NS
)
_PARAMS = pltpu.CompilerParams(
    needs_layout_passes=False, use_tc_tiling_on_sc=False
)


def _wid():
  return lax.axis_index("s") * NC + lax.axis_index("c")


NPW = 320
SPACING = 320
CH = 4
STRIDE_A = K0 + 1
ROWS_A = CH * STRIDE_A
NCH_A = NPW // CH
NBLK = D // 32


def _agg_all_body(featsb_hbm, adj_hbm, out_hbm, adj_v, idx_c, buf0, buf1,
                  stage, sem0, sem1):
  wid = _wid()
  base = pl.multiple_of(jnp.minimum(wid * SPACING, N - NPW), 8)
  pltpu.sync_copy(adj_hbm.at[pl.ds(base, NPW)], adj_v)

  iota = lax.iota(jnp.int32, L)
  msk9 = iota < (K0 - L)

  @pl.loop(0, NPW)
  def _(i):
    v0 = adj_v[i, pl.ds(0, L)]
    v1 = adj_v[i, pl.ds(L, L)]
    b26 = i * STRIDE_A
    plsc.store_scatter(idx_c, [b26 + iota], v0)
    plsc.store_scatter(idx_c, [b26 + L + iota], v1, mask=msk9)
    plsc.store_scatter(idx_c, [jnp.zeros((L,), jnp.int32) + (b26 + K0)], v0)

  def start(g, buf, sem):
    pltpu.async_copy(featsb_hbm.at[idx_c.at[pl.ds(g * ROWS_A, ROWS_A)]],
                     buf, sem)

  def wait(buf, sem):
    pltpu.make_async_copy(featsb_hbm.at[pl.ds(0, ROWS_A)], buf, sem).wait()

  inv = jnp.float32(1.0 / K0)

  def compute(g, buf):
    for t in range(CH):
      node = g * CH + t
      row0 = STRIDE_A * t

      def load_row(row):
        out = []
        for kb in range(NBLK):
          a, b = plsc.unpack(buf[row, pl.ds(32 * kb, 32)],
                             format=plsc.PackFormat.INTERLEAVED)
          out.extend((a, b))
        return tuple(out)

      accs = load_row(row0)

      def body(r, accs, _row0=row0):
        vs = load_row(_row0 + r)
        return tuple(a + v for a, v in zip(accs, vs))

      accs = pl.loop(1, K0, init_carry=accs, unroll=4)(body)
      for kb in range(NBLK):
        stage[node, pl.ds(32 * kb, 16)] = accs[2 * kb] * inv
        stage[node, pl.ds(32 * kb + 16, 16)] = accs[2 * kb + 1] * inv

  start(0, buf0, sem0)

  @pl.loop(0, NCH_A // 2)
  def _(gp):
    g0 = gp * 2
    start(g0 + 1, buf1, sem1)
    wait(buf0, sem0)
    compute(g0, buf0)
    start((g0 + 2) % NCH_A, buf0, sem0)
    wait(buf1, sem1)
    compute(g0 + 1, buf1)

  wait(buf0, sem0)
  pltpu.sync_copy(stage, out_hbm.at[pl.ds(base, NPW)])


_agg_all = functools.partial(
    pl.kernel,
    out_type=jax.ShapeDtypeStruct((N, D), jnp.float32),
    mesh=_MESH,
    compiler_params=_PARAMS,
    scratch_types=[
        pltpu.VMEM((NPW, MAXNB), jnp.int32),
        pltpu.VMEM((NPW * STRIDE_A,), jnp.int32),
        pltpu.VMEM((ROWS_A, D), jnp.bfloat16),
        pltpu.VMEM((ROWS_A, D), jnp.bfloat16),
        pltpu.VMEM((NPW, D), jnp.float32),
        pltpu.SemaphoreType.DMA,
        pltpu.SemaphoreType.DMA,
    ],
)(_agg_all_body)


JB = B // NW
NCH_B = B // L
G_C = 8
KIDX = K1 * G_C
NG_C = JB // G_C
HBLK = H // 32


def _batch_body(x_hbm, nodes_hbm, adj_hbm, aggall_hbm, hnode_hbm,
                xdedup_hbm, agg0_hbm, agg1_hbm,
                nodes_v, dbuf, prevscr, first_v, xrows, adj_rows, idx_c,
                rows0, rows1, agg0buf, agg1st,
                semx, sema, semb, sem0, sem1):
  wid = _wid()
  jbase = pl.multiple_of(wid * JB, 8)
  pltpu.sync_copy(nodes_hbm, nodes_v)
  ndslice = nodes_v.at[pl.ds(jbase, JB)]
  pltpu.async_copy(adj_hbm.at[ndslice], adj_rows, sema)
  pltpu.async_copy(aggall_hbm.at[ndslice], agg0buf, semb)

  iota = lax.iota(jnp.int32, L)
  prevscr[pl.ds(0, L)] = jnp.full((L,), -1, jnp.int32)

  @pl.loop(0, NCH_B)
  def _(i):
    c = (NCH_B - 1) - i
    j = c * L + iota
    nd = nodes_v[pl.ds(c * L, L)]
    key = nd * B + j
    sk, sv = plsc.sort_key_val(key, j)
    snode = lax.shift_right_logical(sk, 12)
    plsc.store_scatter(prevscr, [iota + 1], snode)
    prev = prevscr[pl.ds(0, L)]
    head = (iota == 0) | (snode != prev)
    plsc.store_scatter(dbuf, [snode], sv, mask=head)

  for q in range(JB // L):
    nd16 = nodes_v[pl.ds(jbase + q * L, L)]
    first_v[pl.ds(q * L, L)] = plsc.load_gather(dbuf, [nd16])
  pltpu.async_copy(x_hbm.at[first_v], xrows, semx)

  msk10 = iota < K1
  pltpu.make_async_copy(adj_hbm.at[pl.ds(0, JB)], adj_rows, sema).wait()

  @pl.loop(0, JB)
  def _(j):
    v = adj_rows[j, pl.ds(0, L)]
    plsc.store_scatter(idx_c, [j * K1 + iota], v, mask=msk10)

  pltpu.make_async_copy(aggall_hbm.at[pl.ds(0, JB)], agg0buf, semb).wait()
  pltpu.sync_copy(agg0buf, agg0_hbm.at[pl.ds(jbase, JB)])
  pltpu.make_async_copy(x_hbm.at[pl.ds(0, JB)], xrows, semx).wait()
  pltpu.sync_copy(xrows, xdedup_hbm.at[pl.ds(jbase, JB)])

  inv = jnp.float32(1.0 / K1)

  def start(g, buf, sem_):
    pltpu.async_copy(hnode_hbm.at[idx_c.at[pl.ds(g * KIDX, KIDX)]],
                     buf, sem_)

  def wait(buf, sem_):
    pltpu.make_async_copy(hnode_hbm.at[pl.ds(0, KIDX)], buf, sem_).wait()

  def compute(g, buf):
    for t in range(G_C):
      j = g * G_C + t
      row0 = t * K1

      def load_row(row):
        out = []
        for kb in range(HBLK):
          a, b = plsc.unpack(buf[row, pl.ds(32 * kb, 32)],
                             format=plsc.PackFormat.INTERLEAVED)
          out.extend((a, b))
        return tuple(out)

      accs = load_row(row0)

      def body(r, accs, _row0=row0):
        vs = load_row(_row0 + r)
        return tuple(a + v for a, v in zip(accs, vs))

      accs = pl.loop(1, K1, init_carry=accs, unroll=3)(body)
      for kb in range(HBLK):
        agg1st[j, pl.ds(32 * kb, 16)] = accs[2 * kb] * inv
        agg1st[j, pl.ds(32 * kb + 16, 16)] = accs[2 * kb + 1] * inv

  start(0, rows0, sem0)

  @pl.loop(0, NG_C // 2)
  def _(gp):
    g0 = gp * 2
    start(g0 + 1, rows1, sem1)
    wait(rows0, sem0)
    compute(g0, rows0)
    start((g0 + 2) % NG_C, rows0, sem0)
    wait(rows1, sem1)
    compute(g0 + 1, rows1)

  wait(rows0, sem0)
  pltpu.sync_copy(agg1st, agg1_hbm.at[pl.ds(jbase, JB)])


_batch = functools.partial(
    pl.kernel,
    out_type=(jax.ShapeDtypeStruct((B, D), jnp.float32),
              jax.ShapeDtypeStruct((B, D), jnp.float32),
              jax.ShapeDtypeStruct((B, H), jnp.float32)),
    mesh=_MESH,
    compiler_params=_PARAMS,
    scratch_types=[
        pltpu.VMEM((B,), jnp.int32),
        pltpu.VMEM((N,), jnp.int32),
        pltpu.VMEM((2 * L,), jnp.int32),
        pltpu.VMEM((JB,), jnp.int32),
        pltpu.VMEM((JB, D), jnp.float32),
        pltpu.VMEM((JB, MAXNB), jnp.int32),
        pltpu.VMEM((JB * K1,), jnp.int32),
        pltpu.VMEM((KIDX, H), jnp.bfloat16),
        pltpu.VMEM((KIDX, H), jnp.bfloat16),
        pltpu.VMEM((JB, D), jnp.float32),
        pltpu.VMEM((JB, H), jnp.float32),
        pltpu.SemaphoreType.DMA,
        pltpu.SemaphoreType.DMA,
        pltpu.SemaphoreType.DMA,
        pltpu.SemaphoreType.DMA,
        pltpu.SemaphoreType.DMA,
    ],
)(_batch_body)


def _tc1_body(xa_ref, xb_ref, wa_ref, wb_ref, b_ref, o_ref):
  xa = xa_ref[...].astype(jnp.bfloat16)
  xb = xb_ref[...].astype(jnp.bfloat16)
  o_ref[...] = jax.nn.relu(
      jnp.dot(xa, wa_ref[...], preferred_element_type=jnp.float32)
      + jnp.dot(xb, wb_ref[...], preferred_element_type=jnp.float32)
      + b_ref[...]).astype(o_ref.dtype)


def _tc_linear(xa, xb, waT, wbT, bias, block_rows, out_dtype):
  rows = xa.shape[0]
  ka = xa.shape[1]
  kb = xb.shape[1]
  h = waT.shape[1]
  grid = rows // block_rows
  return pl.pallas_call(
      _tc1_body,
      grid=(grid,),
      in_specs=[
          pl.BlockSpec((block_rows, ka), lambda i: (i, 0)),
          pl.BlockSpec((block_rows, kb), lambda i: (i, 0)),
          pl.BlockSpec((ka, h), lambda i: (0, 0)),
          pl.BlockSpec((kb, h), lambda i: (0, 0)),
          pl.BlockSpec((1, h), lambda i: (0, 0)),
      ],
      out_specs=pl.BlockSpec((block_rows, h), lambda i: (i, 0)),
      out_shape=jax.ShapeDtypeStruct((rows, h), out_dtype),
  )(xa, xb, waT, wbT, bias)


def _tc2_body(xd_ref, a0_ref, a1_ref, w0a_ref, w0b_ref, b0_ref,
              w1a_ref, w1b_ref, b1_ref, o_ref):
  xd = xd_ref[...].astype(jnp.bfloat16)
  a0 = a0_ref[...].astype(jnp.bfloat16)
  a1 = a1_ref[...].astype(jnp.bfloat16)
  h0 = jax.nn.relu(
      jnp.dot(xd, w0a_ref[...], preferred_element_type=jnp.float32)
      + jnp.dot(a0, w0b_ref[...], preferred_element_type=jnp.float32)
      + b0_ref[...]).astype(jnp.bfloat16)
  o_ref[...] = jax.nn.relu(
      jnp.dot(h0, w1a_ref[...], preferred_element_type=jnp.float32)
      + jnp.dot(a1, w1b_ref[...], preferred_element_type=jnp.float32)
      + b1_ref[...])


def _tc_layer2(xd, a0, a1, w0aT, w0bT, b0r, w1aT, w1bT, b1r, block_rows):
  rows = xd.shape[0]
  grid = rows // block_rows
  return pl.pallas_call(
      _tc2_body,
      grid=(grid,),
      in_specs=[
          pl.BlockSpec((block_rows, D), lambda i: (i, 0)),
          pl.BlockSpec((block_rows, D), lambda i: (i, 0)),
          pl.BlockSpec((block_rows, H), lambda i: (i, 0)),
          pl.BlockSpec((D, H), lambda i: (0, 0)),
          pl.BlockSpec((D, H), lambda i: (0, 0)),
          pl.BlockSpec((1, H), lambda i: (0, 0)),
          pl.BlockSpec((H, H), lambda i: (0, 0)),
          pl.BlockSpec((H, H), lambda i: (0, 0)),
          pl.BlockSpec((1, H), lambda i: (0, 0)),
      ],
      out_specs=pl.BlockSpec((block_rows, H), lambda i: (i, 0)),
      out_shape=jax.ShapeDtypeStruct((rows, H), jnp.float32),
  )(xd, a0, a1, w0aT, w0bT, b0r, w1aT, w1bT, b1r)


def _deinterleave_perm(width):
  blocks = []
  for kb in range(width // 32):
    evens = 32 * kb + 2 * np.arange(16)
    blocks.append(np.concatenate([evens, evens + 1]))
  return np.concatenate(blocks)


_PERM128 = _deinterleave_perm(D)
_PERM256 = _deinterleave_perm(H)


def kernel(x, nodes, feats, adj, W0, b0, W1, b1):
  assert feats.shape == (N, D) and nodes.shape == (B,)
  assert adj.shape == (N, MAXNB)
  nodes = nodes.astype(jnp.int32)
  adj32 = adj.astype(jnp.int32)
  featsb = feats.astype(jnp.bfloat16)
  w0aT = W0[:, :D].T.astype(jnp.bfloat16)
  w0bT = W0[:, D:].T[_PERM128].astype(jnp.bfloat16)
  w1aT = W1[:, :H].T.astype(jnp.bfloat16)
  w1bT = W1[:, H:].T[_PERM256].astype(jnp.bfloat16)
  b0r = b0.reshape(1, H)
  b1r = b1.reshape(1, H)

  agg_all = _agg_all(featsb, adj32)
  h_node = _tc_linear(feats, agg_all, w0aT, w0bT, b0r, block_rows=2000,
                      out_dtype=jnp.bfloat16)
  x_dedup, agg0, agg1 = _batch(x, nodes, adj32, agg_all, h_node)
  out = _tc_layer2(x_dedup, agg0, agg1, w0aT, w0bT, b0r,
                   w1aT, w1bT, b1r, block_rows=1024)
  return out
